# baseline (speedup 1.0000x reference)
import functools

import jax
import jax.numpy as jnp
from jax.experimental import pallas as pl
from jax.experimental.pallas import tpu as pltpu

_BR = 512
_LC = 128


def _select_kernel(y_ref, x_ref, o_ref):
    yb = y_ref[...].reshape(_BR, 1)
    C = x_ref.shape[1]
    acc = jnp.zeros((_BR, _LC), jnp.float32)
    for k in range(0, C, _LC):
        w = min(_LC, C - k)
        x = x_ref[:, k:k + w]
        ids = jax.lax.broadcasted_iota(jnp.int32, (_BR, w), 1) + k
        hit = jnp.where(ids == yb, x, 0.0)
        if w < _LC:
            hit = jnp.pad(hit, ((0, 0), (0, _LC - w)))
        acc = acc + hit
    o_ref[...] = jnp.sum(acc, axis=1)


def kernel(logits, y):
    B, C = logits.shape
    y32 = y.astype(jnp.int32)
    grid = (B // _BR,)
    return pl.pallas_call(
        _select_kernel,
        grid=grid,
        in_specs=[
            pl.BlockSpec((_BR,), lambda i: (i,)),
            pl.BlockSpec((_BR, C), lambda i: (i, 0)),
        ],
        out_specs=pl.BlockSpec((_BR,), lambda i: (i,)),
        out_shape=jax.ShapeDtypeStruct((B,), jnp.float32),
    )(y32, logits)


# speedup vs baseline: 1.1005x; 1.1005x over previous
import functools

import jax
import jax.numpy as jnp
from jax.experimental import pallas as pl
from jax.experimental.pallas import tpu as pltpu

_BR = 256
_NS = 4
_LC = 128


def _select_block(yb, x):
    BR, C = x.shape
    acc = jnp.zeros((BR, _LC), jnp.float32)
    for k in range(0, C, _LC):
        w = min(_LC, C - k)
        ids = jax.lax.broadcasted_iota(jnp.int32, (BR, w), 1) + k
        hit = jnp.where(ids == yb, x[:, k:k + w], 0.0)
        if w < _LC:
            hit = jnp.pad(hit, ((0, 0), (0, _LC - w)))
        acc = acc + hit
    return jnp.sum(acc, axis=1)


def _select_kernel(y_ref, *refs):
    x_refs = refs[:_NS]
    o_ref = refs[_NS]
    for j in range(_NS):
        yb = y_ref[pl.ds(j * _BR, _BR)].reshape(_BR, 1)
        o_ref[pl.ds(j * _BR, _BR)] = _select_block(yb, x_refs[j][...])


def kernel(logits, y):
    B, C = logits.shape
    y32 = y.astype(jnp.int32)
    grid = (B // (_BR * _NS),)
    x_specs = [
        pl.BlockSpec((_BR, C), functools.partial(lambda j, i: (_NS * i + j, 0), j))
        for j in range(_NS)
    ]
    return pl.pallas_call(
        _select_kernel,
        grid=grid,
        in_specs=[pl.BlockSpec((_BR * _NS,), lambda i: (i,))] + x_specs,
        out_specs=pl.BlockSpec((_BR * _NS,), lambda i: (i,)),
        out_shape=jax.ShapeDtypeStruct((B,), jnp.float32),
    )(y32, *([logits] * _NS))


# speedup vs baseline: 1.1652x; 1.0588x over previous
import functools

import jax
import jax.numpy as jnp
from jax.experimental import pallas as pl
from jax.experimental.pallas import tpu as pltpu

_BR = 2048
_NS = 1
_LC = 128


def _select_block(yb, x):
    BR, C = x.shape
    acc = jnp.zeros((BR, _LC), jnp.float32)
    for k in range(0, C, _LC):
        w = min(_LC, C - k)
        ids = jax.lax.broadcasted_iota(jnp.int32, (BR, w), 1) + k
        hit = jnp.where(ids == yb, x[:, k:k + w], 0.0)
        if w < _LC:
            hit = jnp.pad(hit, ((0, 0), (0, _LC - w)))
        acc = acc + hit
    return jnp.sum(acc, axis=1)


def _select_kernel(y_ref, *refs):
    x_refs = refs[:_NS]
    o_ref = refs[_NS]
    for j in range(_NS):
        yb = y_ref[pl.ds(j * _BR, _BR)].reshape(_BR, 1)
        o_ref[pl.ds(j * _BR, _BR)] = _select_block(yb, x_refs[j][...])


def kernel(logits, y):
    B, C = logits.shape
    y32 = y.astype(jnp.int32)
    grid = (B // (_BR * _NS),)
    x_specs = [
        pl.BlockSpec((_BR, C), functools.partial(lambda j, i: (_NS * i + j, 0), j))
        for j in range(_NS)
    ]
    return pl.pallas_call(
        _select_kernel,
        grid=grid,
        in_specs=[pl.BlockSpec((_BR * _NS,), lambda i: (i,))] + x_specs,
        out_specs=pl.BlockSpec((_BR * _NS,), lambda i: (i,)),
        out_shape=jax.ShapeDtypeStruct((B,), jnp.float32),
    )(y32, *([logits] * _NS))
